# Initial kernel scaffold; baseline (speedup 1.0000x reference)
import jax
import jax.numpy as jnp
from jax.experimental import pallas as pl


def kernel(grid0, grid1, grid2):
    raise NotImplementedError("write your pallas kernel here")


# speedup vs baseline: 17.1324x; 17.1324x over previous
import functools

import numpy as np
import jax
import jax.numpy as jnp
from jax.experimental import pallas as pl
from jax.experimental.pallas import tpu as pltpu

_NCLASSES = 80
_MAX_OUT = 100
_IOU_THR = 0.5
_NEG = np.float32(-1e10)
_N = 6300
_NPAD = 6400
_ROWS, _COLS = 8, 800

_ANCHORS = np.array([
    [[0.024, 0.031], [0.038, 0.072], [0.079, 0.055]],
    [[0.072, 0.147], [0.149, 0.108], [0.142, 0.286]],
    [[0.279, 0.216], [0.375, 0.476], [0.897, 0.784]],
], dtype=np.float32)


def _build_consts():
    gx = np.zeros(_NPAD, np.float32)
    gy = np.zeros(_NPAD, np.float32)
    aw = np.ones(_NPAD, np.float32)
    ah = np.ones(_NPAD, np.float32)
    gsv = np.ones(_NPAD, np.float32)
    vmask = np.zeros(_NPAD, np.float32)
    base = 0
    for gs, anc in ((10, _ANCHORS[2]), (20, _ANCHORS[1]), (40, _ANCHORS[0])):
        n = gs * gs * 3
        ii, jj, aa = np.meshgrid(np.arange(gs), np.arange(gs), np.arange(3),
                                 indexing="ij")
        gx[base:base + n] = jj.ravel().astype(np.float32)
        gy[base:base + n] = ii.ravel().astype(np.float32)
        aw[base:base + n] = anc[aa.ravel(), 0]
        ah[base:base + n] = anc[aa.ravel(), 1]
        gsv[base:base + n] = float(gs)
        base += n
    vmask[:_N] = 1.0
    c = np.stack([gx, gy, aw, ah, gsv, vmask])
    return c.reshape(6, _ROWS, _COLS)

_CONSTS = _build_consts()


def _nms_kernel(feat_ref, const_ref, boxes_ref, scores_ref, cls_ref, nv_ref):
    gx = const_ref[0]
    gy = const_ref[1]
    aw = const_ref[2]
    ah = const_ref[3]
    gsv = const_ref[4]
    vmaskb = const_ref[5] > 0.0

    x = (feat_ref[0] + gx) / gsv
    y = (feat_ref[1] + gy) / gsv
    w = jnp.exp(feat_ref[2]) * aw
    h = jnp.exp(feat_ref[3]) * ah
    conf = feat_ref[4]
    x1 = x - w * 0.5
    x2 = x + w * 0.5
    y1 = y - h * 0.5
    y2 = y + h * 0.5
    area = jnp.maximum(x2 - x1, 0.0) * jnp.maximum(y2 - y1, 0.0)

    best = feat_ref[5]
    bidx = jnp.zeros((_ROWS, _COLS), jnp.float32)
    for k in range(1, _NCLASSES):
        c = feat_ref[5 + k]
        upd = c > best
        best = jnp.where(upd, c, best)
        bidx = jnp.where(upd, jnp.float32(k), bidx)

    cmax_hi = jnp.max(jnp.where(vmaskb, best, jnp.float32(-3e38)))
    cmax_lo = jnp.min(jnp.where(vmaskb, best, jnp.float32(3e38)))
    s0 = conf * jnp.where(conf > 0.0, cmax_hi, cmax_lo)
    s0 = jnp.where((s0 > 0.0) & vmaskb, s0, _NEG)

    fi = (jax.lax.broadcasted_iota(jnp.int32, (_ROWS, _COLS), 0) * _COLS
          + jax.lax.broadcasted_iota(jnp.int32, (_ROWS, _COLS), 1))
    i4 = jax.lax.broadcasted_iota(jnp.int32, (1, 4), 1)

    def body(it, carry):
        s, wx1, wy1, wx2, wy2, wa, wg, nv = carry
        ix1 = jnp.maximum(wx1, x1)
        iy1 = jnp.maximum(wy1, y1)
        ix2 = jnp.minimum(wx2, x2)
        iy2 = jnp.minimum(wy2, y2)
        inter = jnp.maximum(ix2 - ix1, 0.0) * jnp.maximum(iy2 - iy1, 0.0)
        union = wa + area - inter
        upos = union > 0.0
        iou = jnp.where(upos, inter / jnp.where(upos, union, 1.0), 0.0)
        kill = (iou > _IOU_THR) | (fi == wg)
        s = jnp.where(kill, _NEG, s)
        m_ = jnp.max(s)
        g = jnp.min(jnp.where(s == m_, fi, jnp.int32(_NPAD)))
        mask = fi == g
        pick = lambda v: jnp.sum(jnp.where(mask, v, 0.0))
        nx1 = pick(x1)
        ny1 = pick(y1)
        nx2 = pick(x2)
        ny2 = pick(y2)
        na = pick(area)
        ncls = pick(bidx)
        valid = m_ > _NEG * 0.5
        row4 = jnp.where(i4 == 0, nx1,
               jnp.where(i4 == 1, ny1,
               jnp.where(i4 == 2, nx2, ny2)))
        boxes_ref[pl.ds(it, 1), :] = jnp.where(valid, row4, 0.0)
        scores_ref[pl.ds(it, 1), :] = jnp.where(valid, m_, 0.0).reshape(1, 1)
        cls_ref[pl.ds(it, 1), :] = jnp.where(valid, ncls, 0.0).reshape(1, 1)
        return (s, nx1, ny1, nx2, ny2, na, g, nv + valid.astype(jnp.int32))

    zero = jnp.float32(0.0)
    carry = (s0, zero, zero, zero, zero, zero, jnp.int32(-1), jnp.int32(0))
    carry = jax.lax.fori_loop(0, _MAX_OUT, body, carry)
    nv_ref[:, :] = carry[-1].reshape(1, 1)


@jax.jit
def kernel(grid0, grid1, grid2):
    parts = [grid0.reshape(-1, 85), grid1.reshape(-1, 85),
             grid2.reshape(-1, 85)]
    allf = jnp.concatenate(parts, axis=0)
    allf = jnp.pad(allf, ((0, _NPAD - _N), (0, 0)))
    feat = allf.T.reshape(85, _ROWS, _COLS)
    consts = jnp.asarray(_CONSTS)

    boxes, scores, clsf, nv = pl.pallas_call(
        _nms_kernel,
        out_shape=[
            jax.ShapeDtypeStruct((_MAX_OUT, 4), jnp.float32),
            jax.ShapeDtypeStruct((_MAX_OUT, 1), jnp.float32),
            jax.ShapeDtypeStruct((_MAX_OUT, 1), jnp.float32),
            jax.ShapeDtypeStruct((1, 1), jnp.int32),
        ],
        in_specs=[
            pl.BlockSpec(memory_space=pltpu.VMEM),
            pl.BlockSpec(memory_space=pltpu.VMEM),
        ],
        out_specs=[
            pl.BlockSpec(memory_space=pltpu.VMEM),
            pl.BlockSpec(memory_space=pltpu.VMEM),
            pl.BlockSpec(memory_space=pltpu.VMEM),
            pl.BlockSpec(memory_space=pltpu.VMEM),
        ],
    )(feat, consts)

    return (boxes[None], scores[:, 0][None],
            clsf[:, 0].astype(jnp.int32)[None], nv.reshape(1))
